# baseline (device time: 68471 ns/iter reference)
import jax
import jax.numpy as jnp
from jax import lax
from jax.experimental import pallas as pl
from jax.experimental.pallas import tpu as pltpu

N_DEV = 16
NP = 4
NZ = 4


def kernel(x, w_mat, scale_x, scale_w):
    m_total, k_per = x.shape
    k_total, n = w_mat.shape
    m_per = m_total // N_DEV

    x4 = x.reshape(NZ, NP, m_per, k_per)

    def body(x_ref, w_ref, sx_ref, sw_ref, out_ref, r1, r2, ss1, rs1, ss2, rs2):
        me = lax.axis_index("i")
        p = lax.rem(me, NP)
        z = lax.div(me, NP)

        r1[p] = x_ref[:, p]

        sends1 = []
        for jj in range(1, NP):
            pp = lax.rem(p + jj, NP)
            rdma = pltpu.make_async_remote_copy(
                src_ref=x_ref.at[:, pp],
                dst_ref=r1.at[p],
                send_sem=ss1.at[jj],
                recv_sem=rs1.at[p],
                device_id=(NP * z + pp,),
                device_id_type=pl.DeviceIdType.MESH,
            )
            rdma.start()
            sends1.append(rdma)
        for jj in range(1, NP):
            pp = lax.rem(p + jj, NP)
            recv = pltpu.make_async_remote_copy(
                src_ref=x_ref.at[:, pp],
                dst_ref=r1.at[pp],
                send_sem=ss1.at[0],
                recv_sem=rs1.at[pp],
                device_id=(NP * z + pp,),
                device_id_type=pl.DeviceIdType.MESH,
            )
            recv.wait_recv()

        sends2 = []
        for jj in range(1, NZ):
            zz = lax.rem(z + jj, NZ)
            rdma = pltpu.make_async_remote_copy(
                src_ref=r1.at[:, zz],
                dst_ref=r2.at[z],
                send_sem=ss2.at[jj],
                recv_sem=rs2.at[z],
                device_id=(NP * zz + p,),
                device_id_type=pl.DeviceIdType.MESH,
            )
            rdma.start()
            sends2.append(rdma)

        r2[z] = r1[:, z]

        for jj in range(1, NZ):
            zz = lax.rem(z + jj, NZ)
            recv = pltpu.make_async_remote_copy(
                src_ref=r1.at[:, zz],
                dst_ref=r2.at[zz],
                send_sem=ss2.at[0],
                recv_sem=rs2.at[zz],
                device_id=(NP * zz + p,),
                device_id_type=pl.DeviceIdType.MESH,
            )
            recv.wait_recv()
        for rdma in sends1:
            rdma.wait_send()
        for rdma in sends2:
            rdma.wait_send()

        acc = None
        for k in range(N_DEV):
            xb = r2[k // NP, k % NP].astype(jnp.bfloat16)
            wb = w_ref[k * k_per:(k + 1) * k_per, :].astype(jnp.bfloat16)
            d = jnp.dot(xb, wb, preferred_element_type=jnp.float32)
            acc = d if acc is None else acc + d
        out_ref[:, :] = acc * (sx_ref[0] * sw_ref[0])

    return pl.pallas_call(
        body,
        out_shape=jax.ShapeDtypeStruct((m_per, n), jnp.float32),
        in_specs=[
            pl.BlockSpec(memory_space=pltpu.VMEM),
            pl.BlockSpec(memory_space=pltpu.VMEM),
            pl.BlockSpec(memory_space=pltpu.SMEM),
            pl.BlockSpec(memory_space=pltpu.SMEM),
        ],
        out_specs=pl.BlockSpec(memory_space=pltpu.VMEM),
        scratch_shapes=[
            pltpu.VMEM((NP, NZ, m_per, k_per), jnp.int8),
            pltpu.VMEM((NZ, NP, m_per, k_per), jnp.int8),
            pltpu.SemaphoreType.DMA((NP,)),
            pltpu.SemaphoreType.DMA((NP,)),
            pltpu.SemaphoreType.DMA((NZ,)),
            pltpu.SemaphoreType.DMA((NZ,)),
        ],
        compiler_params=pltpu.CompilerParams(
            vmem_limit_bytes=100 * 1024 * 1024,
        ),
    )(x4, w_mat, scale_x, scale_w)


# device time: 58054 ns/iter; 1.1794x vs baseline; 1.1794x over previous
import jax
import jax.numpy as jnp
from jax import lax
from jax.experimental import pallas as pl
from jax.experimental.pallas import tpu as pltpu

N_DEV = 16


def kernel(x, w_mat, scale_x, scale_w):
    m_total, k_per = x.shape
    k_total, n = w_mat.shape
    m_per = m_total // N_DEV

    def body(x_ref, w_ref, sx_ref, sw_ref, out_ref, comm_ref, send_sems, recv_sems):
        me = lax.axis_index("i")

        comm_ref[:, pl.ds(me * k_per, k_per)] = x_ref[pl.ds(me * m_per, m_per), :]

        sends = []
        for o in range(1, N_DEV):
            dst = lax.rem(me + o, N_DEV)
            rdma = pltpu.make_async_remote_copy(
                src_ref=x_ref.at[pl.ds(dst * m_per, m_per), :],
                dst_ref=comm_ref.at[:, pl.ds(me * k_per, k_per)],
                send_sem=send_sems.at[o],
                recv_sem=recv_sems.at[me],
                device_id=(dst,),
                device_id_type=pl.DeviceIdType.MESH,
            )
            rdma.start()
            sends.append(rdma)

        for o in range(1, N_DEV):
            src = lax.rem(me - o + N_DEV, N_DEV)
            pltpu.make_async_remote_copy(
                src_ref=x_ref.at[pl.ds(src * m_per, m_per), :],
                dst_ref=comm_ref.at[:, pl.ds(src * k_per, k_per)],
                send_sem=send_sems.at[0],
                recv_sem=recv_sems.at[src],
                device_id=(src,),
                device_id_type=pl.DeviceIdType.MESH,
            ).wait_recv()

        acc = jnp.dot(comm_ref[:, :], w_ref[:, :], preferred_element_type=jnp.float32)
        out_ref[:, :] = acc * (sx_ref[0] * sw_ref[0])

        for rdma in sends:
            rdma.wait_send()

    return pl.pallas_call(
        body,
        out_shape=jax.ShapeDtypeStruct((m_per, n), jnp.float32),
        in_specs=[
            pl.BlockSpec(memory_space=pltpu.VMEM),
            pl.BlockSpec(memory_space=pltpu.VMEM),
            pl.BlockSpec(memory_space=pltpu.SMEM),
            pl.BlockSpec(memory_space=pltpu.SMEM),
        ],
        out_specs=pl.BlockSpec(memory_space=pltpu.VMEM),
        scratch_shapes=[
            pltpu.VMEM((m_per, k_total), jnp.int8),
            pltpu.SemaphoreType.DMA((N_DEV,)),
            pltpu.SemaphoreType.DMA((N_DEV,)),
        ],
        compiler_params=pltpu.CompilerParams(
            vmem_limit_bytes=100 * 1024 * 1024,
        ),
    )(x, w_mat, scale_x, scale_w)


# device time: 55463 ns/iter; 1.2345x vs baseline; 1.0467x over previous
import jax
import jax.numpy as jnp
from jax import lax
from jax.experimental import pallas as pl
from jax.experimental.pallas import tpu as pltpu

N_DEV = 16
NP = 4


def kernel(x, w_mat, scale_x, scale_w):
    m_total, k_per = x.shape
    k_total, n = w_mat.shape
    m_per = m_total // N_DEV
    half_k = k_total // 2

    x4 = x.reshape(NP, NP, m_per, k_per)

    def body(x_ref, w_ref, sx_ref, sw_ref, out_ref, slab, ss, rs):
        me = lax.axis_index("i")
        mz, mp = lax.div(me, NP), lax.rem(me, NP)

        slab[mz, mp] = x_ref[mz, mp]

        sends = []
        for o in range(1, N_DEV):
            dst = lax.rem(me + o, N_DEV)
            rdma = pltpu.make_async_remote_copy(
                src_ref=x_ref.at[lax.div(dst, NP), lax.rem(dst, NP)],
                dst_ref=slab.at[mz, mp],
                send_sem=ss.at[o],
                recv_sem=rs.at[me],
                device_id=(dst,),
                device_id_type=pl.DeviceIdType.MESH,
            )
            rdma.start()
            sends.append(rdma)

        scale = sx_ref[0] * sw_ref[0]
        for half in range(2):
            for s in range(half * 8, half * 8 + 8):
                @pl.when(s != me)
                def _(s=s):
                    pltpu.make_async_remote_copy(
                        src_ref=x_ref.at[s // NP, s % NP],
                        dst_ref=slab.at[s // NP, s % NP],
                        send_sem=ss.at[0],
                        recv_sem=rs.at[s],
                        device_id=(s,),
                        device_id_type=pl.DeviceIdType.MESH,
                    ).wait_recv()
            xh = jnp.concatenate(
                [slab[s // NP, s % NP].astype(jnp.bfloat16)
                 for s in range(half * 8, half * 8 + 8)], axis=1)
            d = jnp.dot(xh,
                        w_ref[half * half_k:(half + 1) * half_k, :].astype(jnp.bfloat16),
                        preferred_element_type=jnp.float32)
            if half == 0:
                out_ref[:, :] = d * scale
            else:
                out_ref[:, :] = out_ref[:, :] + d * scale

        for rdma in sends:
            rdma.wait_send()

    return pl.pallas_call(
        body,
        out_shape=jax.ShapeDtypeStruct((m_per, n), jnp.float32),
        in_specs=[
            pl.BlockSpec(memory_space=pltpu.VMEM),
            pl.BlockSpec(memory_space=pltpu.VMEM),
            pl.BlockSpec(memory_space=pltpu.SMEM),
            pl.BlockSpec(memory_space=pltpu.SMEM),
        ],
        out_specs=pl.BlockSpec(memory_space=pltpu.VMEM),
        scratch_shapes=[
            pltpu.VMEM((NP, NP, m_per, k_per), jnp.int8),
            pltpu.SemaphoreType.DMA((N_DEV,)),
            pltpu.SemaphoreType.DMA((N_DEV,)),
        ],
        compiler_params=pltpu.CompilerParams(
            vmem_limit_bytes=100 * 1024 * 1024,
        ),
    )(x4, w_mat, scale_x, scale_w)
